# Initial kernel scaffold; baseline (speedup 1.0000x reference)
import jax
import jax.numpy as jnp
from jax.experimental import pallas as pl


def kernel(pyramid_network, image_height, image_width, gt_boxes, W_rpn, b_rpn, W_box, b_box, W_cls, b_cls):
    raise NotImplementedError("write your pallas kernel here")


# speedup vs baseline: 40.5632x; 40.5632x over previous
import functools

import numpy as np
import jax
import jax.numpy as jnp
from jax import lax
from jax.experimental import pallas as pl
from jax.experimental.pallas import tpu as pltpu
from jax.experimental.pallas import tpu_sc as plsc

_A = 9
_STRIDE = 32
_PRE = 6000
_POST = 2000
_TH = 0.7
_NQ = 1152
_NFLAT = _A * _NQ
_NR = _NFLAT // 128
_NRP = 88
_NS = 6016
_NB = _NS // 128


def _anchor_consts():
    base_size, ratios, scales = 16, (0.5, 1.0, 2.0), (16, 32, 64)
    w = h = float(base_size)
    x_ctr = 0.5 * (w - 1.0)
    y_ctr = 0.5 * (h - 1.0)
    size = w * h
    rows = []
    for r in ratios:
        ws = np.round(np.sqrt(size / r))
        hs = np.round(ws * r)
        for s in scales:
            W2 = ws * s
            H2 = hs * s
            rows.append([x_ctr - 0.5 * (W2 - 1), y_ctr - 0.5 * (H2 - 1),
                         x_ctr + 0.5 * (W2 - 1), y_ctr + 0.5 * (H2 - 1)])
    b = np.array(rows, np.float64)
    aw = b[:, 2] - b[:, 0] + 1.0
    ah = b[:, 3] - b[:, 1] + 1.0
    acx = b[:, 0] + 0.5 * aw
    acy = b[:, 1] + 0.5 * ah
    q = np.arange(_NQ)
    qh, qw = q // 34, q % 34
    ACX = (acx[:, None] + (qw * _STRIDE)[None, :]).astype(np.float32)
    ACY = (acy[:, None] + (qh * _STRIDE)[None, :]).astype(np.float32)
    AW = np.broadcast_to(aw[:, None], (9, _NQ)).astype(np.float32).copy()
    AH = np.broadcast_to(ah[:, None], (9, _NQ)).astype(np.float32).copy()
    qv = (qh < 32) & (qw < 32)
    QV = np.broadcast_to(qv[None, :], (9, _NQ)).astype(np.float32).copy()
    ai = (qh * 32 + qw) * 9 + np.arange(9)[:, None]
    slot = np.arange(9 * _NQ).reshape(9, _NQ)
    AI = np.where(qv[None, :], ai, 10**7 + slot).astype(np.int32)
    return ACX, ACY, AW, AH, QV, AI


_ACX, _ACY, _AW, _AH, _QV, _AI = _anchor_consts()
_BOXPERM = np.array([a * 4 + j for j in range(4) for a in range(9)], np.int32)


def _k1_body(xcat, wcat, wc0, wc1, bc0, bc1, wbx, wby, wbw, wbh,
             bbx, bby, bbw, bbh, brpn, acx, acy, aw, ah, qv, bounds,
             ox1, oy1, ox2, oy2, osm):
    y = jnp.dot(wcat[...], xcat[...], preferred_element_type=jnp.float32)
    y = jnp.maximum(y + brpn[...], 0.0)
    ca = jnp.dot(wc0[...], y, preferred_element_type=jnp.float32) + bc0[...]
    cb = jnp.dot(wc1[...], y, preferred_element_type=jnp.float32) + bc1[...]
    m = jnp.maximum(jnp.max(ca, axis=0, keepdims=True),
                    jnp.max(cb, axis=0, keepdims=True))
    ea = jnp.exp(ca - m)
    eb = jnp.exp(cb - m)
    s = jnp.sum(ea, axis=0, keepdims=True) + jnp.sum(eb, axis=0, keepdims=True)
    sc = eb / s
    dx = jnp.dot(wbx[...], y, preferred_element_type=jnp.float32) + bbx[...]
    dy = jnp.dot(wby[...], y, preferred_element_type=jnp.float32) + bby[...]
    dw = jnp.dot(wbw[...], y, preferred_element_type=jnp.float32) + bbw[...]
    dh = jnp.dot(wbh[...], y, preferred_element_type=jnp.float32) + bbh[...]
    acx_, acy_, aw_, ah_ = acx[...], acy[...], aw[...], ah[...]
    pcx = dx * aw_ + acx_
    pcy = dy * ah_ + acy_
    pw = jnp.exp(dw) * aw_
    ph = jnp.exp(dh) * ah_
    xmax = bounds[0, 0]
    ymax = bounds[0, 1]
    x1 = jnp.clip(pcx - 0.5 * pw, 0.0, xmax)
    y1 = jnp.clip(pcy - 0.5 * ph, 0.0, ymax)
    x2 = jnp.clip(pcx + 0.5 * pw, 0.0, xmax)
    y2 = jnp.clip(pcy + 0.5 * ph, 0.0, ymax)
    ws = x2 - x1 + 1.0
    hs = y2 - y1 + 1.0
    valid = (ws >= 16.0) & (hs >= 16.0)
    sm = jnp.where(valid, sc, -1e9)
    sm = jnp.where(qv[...] > 0.5, sm, -2e9)
    ox1[...] = x1
    oy1[...] = y1
    ox2[...] = x2
    oy2[...] = y2
    osm[...] = sm


def _k2_body(s_all, ai_all, s_blk, ai_blk, out):
    rows = []
    for rr in range(8):
        scol = jnp.transpose(s_blk[rr:rr + 1, :])
        aicol = jnp.transpose(ai_blk[rr:rr + 1, :])

        def body(js, acc):
            srow = s_all[pl.ds(js, 1), :]
            airow = ai_all[pl.ds(js, 1), :]
            gt = srow > scol
            eq = (srow == scol) & (airow < aicol)
            return acc + (gt | eq).astype(jnp.int32)

        acc = lax.fori_loop(0, _NRP, body, jnp.zeros((128, 128), jnp.int32))
        rows.append(jnp.transpose(jnp.sum(acc, axis=1, keepdims=True)))
    out[...] = jnp.concatenate(rows, axis=0)


@functools.cache
def _make_k3():
  mesh = plsc.VectorSubcoreMesh(core_axis_name="c", subcore_axis_name="s")

  @functools.partial(
      pl.kernel,
      out_type=[jax.ShapeDtypeStruct((_NS,), jnp.float32)] * 4,
      mesh=mesh,
      scratch_types=[pltpu.VMEM((_NFLAT,), jnp.int32)]
      + [pltpu.VMEM((_NFLAT,), jnp.float32)] * 4
      + [pltpu.VMEM((_NS,), jnp.float32)] * 4,
      compiler_params=pltpu.CompilerParams(needs_layout_passes=False),
  )
  def _k3(rank_h, x1_h, y1_h, x2_h, y2_h, ox1_h, oy1_h, ox2_h, oy2_h,
          rank_v, x1_v, y1_v, x2_v, y2_v, s1_v, s2_v, s3_v, s4_v):
    wid = lax.axis_index("s") * 2 + lax.axis_index("c")

    @pl.when(wid == 0)
    def _():
        pltpu.sync_copy(rank_h, rank_v)
        pltpu.sync_copy(x1_h, x1_v)
        pltpu.sync_copy(y1_h, y1_v)
        pltpu.sync_copy(x2_h, x2_v)
        pltpu.sync_copy(y2_h, y2_v)

        def body(i, c):
            r = rank_v[pl.ds(i * 16, 16)]
            ok = r < _NS
            rc = jnp.minimum(r, _NS - 1)
            plsc.store_scatter(s1_v, [rc], x1_v[pl.ds(i * 16, 16)], mask=ok)
            plsc.store_scatter(s2_v, [rc], y1_v[pl.ds(i * 16, 16)], mask=ok)
            plsc.store_scatter(s3_v, [rc], x2_v[pl.ds(i * 16, 16)], mask=ok)
            plsc.store_scatter(s4_v, [rc], y2_v[pl.ds(i * 16, 16)], mask=ok)
            return c

        lax.fori_loop(0, _NFLAT // 16, body, 0, unroll=4)
        pltpu.sync_copy(s1_v, ox1_h)
        pltpu.sync_copy(s2_v, oy1_h)
        pltpu.sync_copy(s3_v, ox2_h)
        pltpu.sync_copy(s4_v, oy2_h)

  return _k3


def _k4_body(x1, y1, x2, y2, live, keep):
    i = pl.program_id(0)

    @pl.when(i == 0)
    def _():
        keep[...] = live[...]

    px1 = x1[pl.ds(i, 1), :]
    py1 = y1[pl.ds(i, 1), :]
    px2 = x2[pl.ds(i, 1), :]
    py2 = y2[pl.ds(i, 1), :]
    par = (px2 - px1 + 1.0) * (py2 - py1 + 1.0)
    px1c = jnp.transpose(px1)
    py1c = jnp.transpose(py1)
    px2c = jnp.transpose(px2)
    py2c = jnp.transpose(py2)
    parc = jnp.transpose(par)

    xx1 = jnp.maximum(px1c, px1)
    yy1 = jnp.maximum(py1c, py1)
    xx2 = jnp.minimum(px2c, px2)
    yy2 = jnp.minimum(py2c, py2)
    w = jnp.maximum(0.0, xx2 - xx1 + 1.0)
    h = jnp.maximum(0.0, yy2 - yy1 + 1.0)
    inter = w * h
    iou = inter / (parc + par - inter)
    tri = (lax.broadcasted_iota(jnp.int32, (128, 128), 0)
           < lax.broadcasted_iota(jnp.int32, (128, 128), 1))
    M = ((iou > _TH) & tri).astype(jnp.float32)

    kb0 = keep[pl.ds(i, 1), :]

    def fcond(c):
        it, prev, cur = c
        return (it < 130) & jnp.any(prev != cur)

    def fbody(c):
        it, prev, cur = c
        nxt = kb0 * (jnp.dot(cur, M, preferred_element_type=jnp.float32)
                     == 0.0).astype(jnp.float32)
        return it + 1, cur, nxt

    first = kb0 * (jnp.dot(kb0, M, preferred_element_type=jnp.float32)
                   == 0.0).astype(jnp.float32)
    _, _, kb = lax.while_loop(fcond, fbody, (0, kb0, first))
    keep[pl.ds(i, 1), :] = kb

    def cbody(lr, c):
        lx1 = x1[pl.ds(lr, 1), :]
        ly1 = y1[pl.ds(lr, 1), :]
        lx2 = x2[pl.ds(lr, 1), :]
        ly2 = y2[pl.ds(lr, 1), :]
        lar = (lx2 - lx1 + 1.0) * (ly2 - ly1 + 1.0)
        xx1 = jnp.maximum(px1c, lx1)
        yy1 = jnp.maximum(py1c, ly1)
        xx2 = jnp.minimum(px2c, lx2)
        yy2 = jnp.minimum(py2c, ly2)
        w = jnp.maximum(0.0, xx2 - xx1 + 1.0)
        h = jnp.maximum(0.0, yy2 - yy1 + 1.0)
        inter = w * h
        iou = inter / (parc + lar - inter)
        Mx = (iou > _TH).astype(jnp.float32)
        supp = jnp.dot(kb, Mx, preferred_element_type=jnp.float32)
        keep[pl.ds(lr, 1), :] = (keep[pl.ds(lr, 1), :]
                                 * (supp == 0.0).astype(jnp.float32))
        return c

    lax.fori_loop(i + 1, _NB, cbody, 0)


@functools.cache
def _make_k5():
  mesh = plsc.VectorSubcoreMesh(core_axis_name="c", subcore_axis_name="s")

  @functools.partial(
      pl.kernel,
      out_type=[jax.ShapeDtypeStruct((_POST,), jnp.float32)] * 4,
      mesh=mesh,
      scratch_types=[pltpu.VMEM((_NS,), jnp.float32)] * 5
      + [pltpu.VMEM((_POST,), jnp.float32)] * 4,
      compiler_params=pltpu.CompilerParams(needs_layout_passes=False),
  )
  def _k5(keep_h, x1_h, y1_h, x2_h, y2_h, o1_h, o2_h, o3_h, o4_h,
          kv, xv, yv, zv, wv, q1, q2, q3, q4):
    wid = lax.axis_index("s") * 2 + lax.axis_index("c")

    @pl.when(wid == 0)
    def _():
        pltpu.sync_copy(keep_h, kv)
        pltpu.sync_copy(x1_h, xv)
        pltpu.sync_copy(y1_h, yv)
        pltpu.sync_copy(x2_h, zv)
        pltpu.sync_copy(y2_h, wv)

        vx = xv[pl.ds(0, 16)]
        vy = yv[pl.ds(0, 16)]
        vz = zv[pl.ds(0, 16)]
        vw = wv[pl.ds(0, 16)]
        f1 = jnp.full((16,), vx[0], jnp.float32)
        f2 = jnp.full((16,), vy[0], jnp.float32)
        f3 = jnp.full((16,), vz[0], jnp.float32)
        f4 = jnp.full((16,), vw[0], jnp.float32)

        def fill(i, c):
            q1[pl.ds(i * 16, 16)] = f1
            q2[pl.ds(i * 16, 16)] = f2
            q3[pl.ds(i * 16, 16)] = f3
            q4[pl.ds(i * 16, 16)] = f4
            return c

        lax.fori_loop(0, _POST // 16, fill, 0, unroll=4)

        def body(i, cnt):
            k = kv[pl.ds(i * 16, 16)]
            m = k > 0.5
            mi = m.astype(jnp.int32)
            cs = plsc.cumsum(mi)
            pos = cnt + cs - 1
            ok = m & (pos < _POST)
            posc = jnp.minimum(jnp.maximum(pos, 0), _POST - 1)
            plsc.store_scatter(q1, [posc], xv[pl.ds(i * 16, 16)], mask=ok)
            plsc.store_scatter(q2, [posc], yv[pl.ds(i * 16, 16)], mask=ok)
            plsc.store_scatter(q3, [posc], zv[pl.ds(i * 16, 16)], mask=ok)
            plsc.store_scatter(q4, [posc], wv[pl.ds(i * 16, 16)], mask=ok)
            return cnt + jnp.sum(mi)

        lax.fori_loop(0, _NS // 16, body, 0, unroll=4)
        pltpu.sync_copy(q1, o1_h)
        pltpu.sync_copy(q2, o2_h)
        pltpu.sync_copy(q3, o3_h)
        pltpu.sync_copy(q4, o4_h)

  return _k5


def kernel(pyramid_network, image_height, image_width, gt_boxes,
           W_rpn, b_rpn, W_box, b_box, W_cls, b_cls):
    f32 = jnp.float32
    feats = pyramid_network[3].reshape(256, 32, 32)
    xpad = jnp.pad(feats, ((0, 0), (1, 1), (1, 1))).reshape(256, 34 * 34)
    xpad = jnp.pad(xpad, ((0, 0), (0, 70 + _NQ - 34 * 34)))
    xcat = jnp.concatenate(
        [xpad[:, ky * 34 + kx: ky * 34 + kx + _NQ]
         for ky in range(3) for kx in range(3)], axis=0)
    wcat = jnp.transpose(W_rpn, (0, 2, 3, 1)).reshape(256, 2304)
    wc = W_cls.reshape(18, 256)
    wc0, wc1 = wc[0:9], wc[9:18]
    bc0, bc1 = b_cls[0:9].reshape(9, 1), b_cls[9:18].reshape(9, 1)
    wb = W_box.reshape(36, 256)
    wbx, wby, wbw, wbh = wb[0::4], wb[1::4], wb[2::4], wb[3::4]
    bbx, bby = b_box[0::4].reshape(9, 1), b_box[1::4].reshape(9, 1)
    bbw, bbh = b_box[2::4].reshape(9, 1), b_box[3::4].reshape(9, 1)
    brpn = b_rpn.reshape(256, 1)
    bounds = jnp.stack([(image_width - 1.0).astype(f32)
                        if hasattr(image_width, "astype")
                        else jnp.asarray(image_width - 1.0, f32),
                        (image_height - 1.0).astype(f32)
                        if hasattr(image_height, "astype")
                        else jnp.asarray(image_height - 1.0, f32)]).reshape(1, 2)

    shp = jax.ShapeDtypeStruct((9, _NQ), f32)
    vspec = pl.BlockSpec(memory_space=pltpu.VMEM)
    x1, y1, x2, y2, sm = pl.pallas_call(
        _k1_body,
        out_shape=[shp] * 5,
        in_specs=[vspec] * 20 + [pl.BlockSpec(memory_space=pltpu.SMEM)],
        out_specs=[vspec] * 5,
    )(xcat, wcat, wc0, wc1, bc0, bc1, wbx, wby, wbw, wbh,
      bbx, bby, bbw, bbh, brpn,
      jnp.asarray(_ACX), jnp.asarray(_ACY), jnp.asarray(_AW), jnp.asarray(_AH),
      jnp.asarray(_QV), bounds)

    s88 = jnp.concatenate(
        [sm.reshape(_NR, 128),
         jnp.full((_NRP - _NR, 128), -3e9, f32)], axis=0)
    ai88 = jnp.asarray(np.concatenate(
        [_AI.reshape(_NR, 128),
         2 * 10**7 + np.arange((_NRP - _NR) * 128, dtype=np.int32)
         .reshape(_NRP - _NR, 128)], axis=0))
    rank = pl.pallas_call(
        _k2_body,
        grid=(_NRP // 8,),
        out_shape=jax.ShapeDtypeStruct((_NRP, 128), jnp.int32),
        in_specs=[
            pl.BlockSpec((_NRP, 128), lambda c: (0, 0)),
            pl.BlockSpec((_NRP, 128), lambda c: (0, 0)),
            pl.BlockSpec((8, 128), lambda c: (c, 0)),
            pl.BlockSpec((8, 128), lambda c: (c, 0)),
        ],
        out_specs=pl.BlockSpec((8, 128), lambda c: (c, 0)),
    )(s88, ai88, s88, ai88)

    sx1, sy1, sx2, sy2 = _make_k3()(rank[:_NR].reshape(_NFLAT),
                                    x1.reshape(_NFLAT), y1.reshape(_NFLAT),
                                    x2.reshape(_NFLAT), y2.reshape(_NFLAT))

    live = jnp.asarray((np.arange(_NS) < _PRE).reshape(_NB, 128)
                       .astype(np.float32))
    keep = pl.pallas_call(
        _k4_body,
        grid=(_NB,),
        out_shape=jax.ShapeDtypeStruct((_NB, 128), f32),
        in_specs=[pl.BlockSpec((_NB, 128), lambda c: (0, 0))] * 5,
        out_specs=pl.BlockSpec((_NB, 128), lambda c: (0, 0)),
    )(sx1.reshape(_NB, 128), sy1.reshape(_NB, 128),
      sx2.reshape(_NB, 128), sy2.reshape(_NB, 128), live)

    o1, o2, o3, o4 = _make_k5()(keep.reshape(_NS), sx1, sy1, sx2, sy2)
    return jnp.stack([jnp.zeros((_POST,), f32), o1, o2, o3, o4], axis=1)
